# baseline (speedup 1.0000x reference)
import functools

import jax
import jax.numpy as jnp
from jax import lax
from jax.experimental import pallas as pl
from jax.experimental.pallas import tpu as pltpu
from jax.experimental.pallas import tpu_sc as plsc

NC = 2
NS = 16
L = 16
NW = NC * NS
B = 16384
D = 128
BPW = B // NW
CH = 64
NCHUNK = BPW // CH
RU = 8


def _sc_body(users, items, w_u, w_i, w_uc, w_ic,
             out_u, out_i, out_cu, out_ci, out_x,
             idx_u, idx_i,
             buf_u0, buf_i0, buf_uc0, buf_ic0,
             buf_u1, buf_i1, buf_uc1, buf_ic1,
             xbuf, *sems):
    wid = lax.axis_index("s") * NC + lax.axis_index("c")
    base = wid * BPW
    pltpu.sync_copy(users.at[pl.ds(base, BPW)], idx_u)
    pltpu.sync_copy(items.at[pl.ds(base, BPW)], idx_i)

    bufs = [(buf_u0, buf_i0, buf_uc0, buf_ic0),
            (buf_u1, buf_i1, buf_uc1, buf_ic1)]
    sg = [sems[0:4], sems[4:8]]
    sw = [sems[8:12], sems[12:16]]
    outs = (out_u, out_i, out_cu, out_ci)

    def fire_gathers(c, s):
        iu = idx_u.at[pl.ds(c * CH, CH)]
        ii = idx_i.at[pl.ds(c * CH, CH)]
        bu, bi, buc, bic = bufs[s]
        pltpu.async_copy(w_u.at[iu], bu, sg[s][0])
        pltpu.async_copy(w_i.at[ii], bi, sg[s][1])
        pltpu.async_copy(w_uc.at[iu], buc, sg[s][2])
        pltpu.async_copy(w_ic.at[ii], bic, sg[s][3])

    def wait_gather(s, t):
        pltpu.make_async_copy(w_u.at[idx_u.at[pl.ds(0, CH)]],
                              bufs[s][t], sg[s][t]).wait()

    def fire_wb(c, s, t):
        off = base + c * CH
        pltpu.async_copy(bufs[s][t], outs[t].at[pl.ds(off, CH)], sw[s][t])

    def wait_wb(s, t):
        pltpu.make_async_copy(bufs[s][t],
                              outs[t].at[pl.ds(base, CH)], sw[s][t]).wait()

    def compute_cross(c, s):
        buc, bic = bufs[s][2], bufs[s][3]

        def grp(g, vec):
            lanes = lax.iota(jnp.int32, L)
            lane_base = (g % 2) * RU
            for k in range(RU):
                r = g * RU + k
                acc = buc[r, pl.ds(0, L)] * bic[r, pl.ds(0, L)]
                for j in range(1, D // L):
                    acc = acc + (buc[r, pl.ds(j * L, L)]
                                 * bic[r, pl.ds(j * L, L)])
                vec = jnp.where(lanes == lane_base + k, jnp.sum(acc), vec)
            xbuf[pl.ds(c * CH + (g // 2) * L, L)] = vec
            return vec * jnp.where(g % 2 == 1, 0.0, 1.0)

        lax.fori_loop(0, CH // RU, grp, jnp.zeros((L,), jnp.float32))

    fire_gathers(0, 0)
    fire_gathers(1, 1)
    for t in (2, 3):
        wait_gather(0, t)
        fire_wb(0, 0, t)
    compute_cross(0, 0)
    for t in (0, 1):
        wait_gather(0, t)
        fire_wb(0, 0, t)

    def pair(it, carry):
        c0 = 2 * it
        c1 = c0 + 1
        for t in range(4):
            wait_wb(0, t)
        fire_gathers(c0, 0)
        for t in (2, 3):
            wait_gather(1, t)
            fire_wb(c0 - 1, 1, t)
        compute_cross(c0 - 1, 1)
        for t in (0, 1):
            wait_gather(1, t)
            fire_wb(c0 - 1, 1, t)
        for t in range(4):
            wait_wb(1, t)
        fire_gathers(c1, 1)
        for t in (2, 3):
            wait_gather(0, t)
            fire_wb(c0, 0, t)
        compute_cross(c0, 0)
        for t in (0, 1):
            wait_gather(0, t)
            fire_wb(c0, 0, t)
        return carry

    lax.fori_loop(1, NCHUNK // 2, pair, 0)

    for t in (2, 3):
        wait_gather(1, t)
        fire_wb(NCHUNK - 1, 1, t)
    compute_cross(NCHUNK - 1, 1)
    for t in (0, 1):
        wait_gather(1, t)
        fire_wb(NCHUNK - 1, 1, t)
    for s in range(2):
        for t in range(4):
            wait_wb(s, t)
    pltpu.sync_copy(xbuf, out_x.at[pl.ds(base, BPW)])


_mesh = plsc.VectorSubcoreMesh(core_axis_name="c", subcore_axis_name="s")

_sc_call = functools.partial(
    pl.kernel,
    out_type=(
        jax.ShapeDtypeStruct((B, D), jnp.float32),
        jax.ShapeDtypeStruct((B, D), jnp.float32),
        jax.ShapeDtypeStruct((B, D), jnp.float32),
        jax.ShapeDtypeStruct((B, D), jnp.float32),
        jax.ShapeDtypeStruct((B,), jnp.float32),
    ),
    mesh=_mesh,
    compiler_params=pltpu.CompilerParams(
        needs_layout_passes=False, use_tc_tiling_on_sc=False),
    scratch_types=(
        [pltpu.VMEM((BPW,), jnp.int32)] * 2
        + [pltpu.VMEM((CH, D), jnp.float32)] * 8
        + [pltpu.VMEM((BPW,), jnp.float32)]
        + [pltpu.SemaphoreType.DMA] * 16
    ),
)(_sc_body)


@jax.jit
def kernel(users, items, W_user, W_item, W_user_cross, W_item_cross):
    out_u, out_i, out_cu, out_ci, out_x = _sc_call(
        users, items, W_user, W_item, W_user_cross, W_item_cross)
    return out_u, out_i, out_cu, out_ci, out_x.reshape(B, 1)


# speedup vs baseline: 1.0652x; 1.0652x over previous
import functools

import jax
import jax.numpy as jnp
from jax import lax
from jax.experimental import pallas as pl
from jax.experimental.pallas import tpu as pltpu
from jax.experimental.pallas import tpu_sc as plsc

NC = 2
NS = 16
L = 16
NW = NC * NS
B = 16384
D = 128
BPW = B // NW
CH = 64
NCHUNK = BPW // CH
RU = 4


def _sc_body(users, items, w_u, w_i, w_uc, w_ic,
             out_u, out_i, out_cu, out_ci, out_x,
             idx_u, idx_i,
             buf_u0, buf_i0, buf_uc0, buf_ic0,
             buf_u1, buf_i1, buf_uc1, buf_ic1,
             xbuf, *sems):
    wid = lax.axis_index("s") * NC + lax.axis_index("c")
    base = wid * BPW
    pltpu.sync_copy(users.at[pl.ds(base, BPW)], idx_u)
    pltpu.sync_copy(items.at[pl.ds(base, BPW)], idx_i)

    bufs = [(buf_u0, buf_i0, buf_uc0, buf_ic0),
            (buf_u1, buf_i1, buf_uc1, buf_ic1)]
    sg = [sems[0:4], sems[4:8]]
    sw = [sems[8:12], sems[12:16]]
    outs = (out_u, out_i, out_cu, out_ci)

    def fire_gathers(c, s):
        iu = idx_u.at[pl.ds(c * CH, CH)]
        ii = idx_i.at[pl.ds(c * CH, CH)]
        bu, bi, buc, bic = bufs[s]
        pltpu.async_copy(w_u.at[iu], bu, sg[s][0])
        pltpu.async_copy(w_i.at[ii], bi, sg[s][1])
        pltpu.async_copy(w_uc.at[iu], buc, sg[s][2])
        pltpu.async_copy(w_ic.at[ii], bic, sg[s][3])

    def wait_gather(s, t):
        pltpu.make_async_copy(w_u.at[idx_u.at[pl.ds(0, CH)]],
                              bufs[s][t], sg[s][t]).wait()

    def fire_wb(c, s, t):
        off = base + c * CH
        pltpu.async_copy(bufs[s][t], outs[t].at[pl.ds(off, CH)], sw[s][t])

    def wait_wb(s, t):
        pltpu.make_async_copy(bufs[s][t],
                              outs[t].at[pl.ds(base, CH)], sw[s][t]).wait()

    def compute_cross(c, s):
        buc, bic = bufs[s][2], bufs[s][3]

        def grp(g, vec):
            lanes = lax.iota(jnp.int32, L)
            lane_base = (g % 4) * RU
            for k in range(RU):
                r = g * RU + k
                acc = buc[r, pl.ds(0, L)] * bic[r, pl.ds(0, L)]
                for j in range(1, D // L):
                    acc = acc + (buc[r, pl.ds(j * L, L)]
                                 * bic[r, pl.ds(j * L, L)])
                vec = jnp.where(lanes == lane_base + k, jnp.sum(acc), vec)
            xbuf[pl.ds(c * CH + (g // 4) * L, L)] = vec
            return vec * jnp.where(g % 4 == 3, 0.0, 1.0)

        lax.fori_loop(0, CH // RU, grp, jnp.zeros((L,), jnp.float32))

    fire_gathers(0, 0)
    fire_gathers(1, 1)
    for t in (2, 3):
        wait_gather(0, t)
        fire_wb(0, 0, t)
    compute_cross(0, 0)
    for t in (0, 1):
        wait_gather(0, t)
        fire_wb(0, 0, t)

    def pair(it, carry):
        c0 = 2 * it
        c1 = c0 + 1
        for t in range(4):
            wait_wb(0, t)
        fire_gathers(c0, 0)
        for t in (2, 3):
            wait_gather(1, t)
            fire_wb(c0 - 1, 1, t)
        compute_cross(c0 - 1, 1)
        for t in (0, 1):
            wait_gather(1, t)
            fire_wb(c0 - 1, 1, t)
        for t in range(4):
            wait_wb(1, t)
        fire_gathers(c1, 1)
        for t in (2, 3):
            wait_gather(0, t)
            fire_wb(c0, 0, t)
        compute_cross(c0, 0)
        for t in (0, 1):
            wait_gather(0, t)
            fire_wb(c0, 0, t)
        return carry

    lax.fori_loop(1, NCHUNK // 2, pair, 0)

    for t in (2, 3):
        wait_gather(1, t)
        fire_wb(NCHUNK - 1, 1, t)
    compute_cross(NCHUNK - 1, 1)
    for t in (0, 1):
        wait_gather(1, t)
        fire_wb(NCHUNK - 1, 1, t)
    for s in range(2):
        for t in range(4):
            wait_wb(s, t)
    pltpu.sync_copy(xbuf, out_x.at[pl.ds(base, BPW)])


_mesh = plsc.VectorSubcoreMesh(core_axis_name="c", subcore_axis_name="s")

_sc_call = functools.partial(
    pl.kernel,
    out_type=(
        jax.ShapeDtypeStruct((B, D), jnp.float32),
        jax.ShapeDtypeStruct((B, D), jnp.float32),
        jax.ShapeDtypeStruct((B, D), jnp.float32),
        jax.ShapeDtypeStruct((B, D), jnp.float32),
        jax.ShapeDtypeStruct((B,), jnp.float32),
    ),
    mesh=_mesh,
    compiler_params=pltpu.CompilerParams(
        needs_layout_passes=False, use_tc_tiling_on_sc=False),
    scratch_types=(
        [pltpu.VMEM((BPW,), jnp.int32)] * 2
        + [pltpu.VMEM((CH, D), jnp.float32)] * 8
        + [pltpu.VMEM((BPW,), jnp.float32)]
        + [pltpu.SemaphoreType.DMA] * 16
    ),
)(_sc_body)


@jax.jit
def kernel(users, items, W_user, W_item, W_user_cross, W_item_cross):
    out_u, out_i, out_cu, out_ci, out_x = _sc_call(
        users, items, W_user, W_item, W_user_cross, W_item_cross)
    return out_u, out_i, out_cu, out_ci, out_x.reshape(B, 1)
